# baseline (speedup 1.0000x reference)
import jax
import jax.numpy as jnp
from jax.experimental import pallas as pl
from jax.experimental.pallas import tpu as pltpu

_Q = 0.7
_scale_cache = {}


def _get_scale(n, dtype):
    key = (n, dtype)
    if key not in _scale_cache:
        mask = jax.random.bernoulli(jax.random.key(42), _Q, (n,))
        _scale_cache[key] = jnp.where(mask, 0.0, 1.0).astype(dtype)[:, None]
    return _scale_cache[key]


def _mask_body(x_ref, s_ref, o_ref):
    o_ref[...] = x_ref[...] * s_ref[...]


def kernel(x):
    n, d = x.shape
    scale = _get_scale(n, x.dtype)
    br = 1000
    return pl.pallas_call(
        _mask_body,
        grid=(n // br,),
        in_specs=[
            pl.BlockSpec((br, d), lambda i: (i, 0)),
            pl.BlockSpec((br, 1), lambda i: (i, 0)),
        ],
        out_specs=pl.BlockSpec((br, d), lambda i: (i, 0)),
        out_shape=jax.ShapeDtypeStruct((n, d), x.dtype),
        compiler_params=pltpu.CompilerParams(
            dimension_semantics=("parallel",),
        ),
    )(x, scale)


# speedup vs baseline: 1.1641x; 1.1641x over previous
import jax
import jax.numpy as jnp
from jax.experimental import pallas as pl
from jax.experimental.pallas import tpu as pltpu

_Q = 0.7
_scale_cache = {}


def _get_scale(n, dtype):
    key = (n, dtype)
    if key not in _scale_cache:
        mask = jax.random.bernoulli(jax.random.key(42), _Q, (n,))
        _scale_cache[key] = jnp.where(mask, 0.0, 1.0).astype(dtype)[:, None]
    return _scale_cache[key]


def _mask_body(x_ref, s_ref, o_ref):
    o_ref[...] = x_ref[...] * s_ref[...]


def kernel(x):
    n, d = x.shape
    scale = _get_scale(n, x.dtype)
    br = 20000
    return pl.pallas_call(
        _mask_body,
        grid=(n // br,),
        in_specs=[
            pl.BlockSpec((br, d), lambda i: (i, 0)),
            pl.BlockSpec((br, 1), lambda i: (i, 0)),
        ],
        out_specs=pl.BlockSpec((br, d), lambda i: (i, 0)),
        out_shape=jax.ShapeDtypeStruct((n, d), x.dtype),
        compiler_params=pltpu.CompilerParams(
            dimension_semantics=("parallel",),
        ),
    )(x, scale)


# speedup vs baseline: 1.1737x; 1.0083x over previous
import jax
import jax.numpy as jnp
from jax.experimental import pallas as pl
from jax.experimental.pallas import tpu as pltpu

_Q = 0.7
_BR = 2000
_NBUF = 8
_scale_cache = {}


def _get_scale(n, dtype):
    key = (n, dtype)
    if key not in _scale_cache:
        mask = jax.random.bernoulli(jax.random.key(42), _Q, (n,))
        _scale_cache[key] = jnp.where(mask, 0.0, 1.0).astype(dtype)[:, None]
    return _scale_cache[key]


def _pipe_body(scale_hbm, x_hbm, o_hbm, inbuf, sbuf, outbuf, insem, ssem, outsem):
    nchunk = x_hbm.shape[0] // _BR

    def in_copy(c):
        slot = c % _NBUF
        return pltpu.make_async_copy(
            x_hbm.at[pl.ds(c * _BR, _BR), :], inbuf.at[slot], insem.at[slot])

    def s_copy(c):
        slot = c % _NBUF
        return pltpu.make_async_copy(
            scale_hbm.at[pl.ds(c * _BR, _BR), :], sbuf.at[slot], ssem.at[slot])

    def out_copy(c):
        slot = c % _NBUF
        return pltpu.make_async_copy(
            outbuf.at[slot], o_hbm.at[pl.ds(c * _BR, _BR), :], outsem.at[slot])

    for c in range(min(_NBUF, nchunk)):
        in_copy(c).start()
        s_copy(c).start()
    for c in range(nchunk):
        slot = c % _NBUF
        in_copy(c).wait()
        s_copy(c).wait()
        if c >= _NBUF:
            out_copy(c - _NBUF).wait()
        outbuf[slot] = inbuf[slot] * sbuf[slot]
        out_copy(c).start()
        if c + _NBUF < nchunk:
            in_copy(c + _NBUF).start()
            s_copy(c + _NBUF).start()
    for c in range(max(0, nchunk - _NBUF), nchunk):
        out_copy(c).wait()


def kernel(x):
    n, d = x.shape
    scale = _get_scale(n, x.dtype)
    return pl.pallas_call(
        _pipe_body,
        in_specs=[
            pl.BlockSpec(memory_space=pl.ANY),
            pl.BlockSpec(memory_space=pl.ANY),
        ],
        out_specs=pl.BlockSpec(memory_space=pl.ANY),
        out_shape=jax.ShapeDtypeStruct((n, d), x.dtype),
        scratch_shapes=[
            pltpu.VMEM((_NBUF, _BR, d), x.dtype),
            pltpu.VMEM((_NBUF, _BR, 1), x.dtype),
            pltpu.VMEM((_NBUF, _BR, d), x.dtype),
            pltpu.SemaphoreType.DMA((_NBUF,)),
            pltpu.SemaphoreType.DMA((_NBUF,)),
            pltpu.SemaphoreType.DMA((_NBUF,)),
        ],
    )(scale, x)


# speedup vs baseline: 1.4731x; 1.2551x over previous
import functools
import numpy as np
import jax
import jax.numpy as jnp
from jax import lax
from jax.experimental import pallas as pl
from jax.experimental.pallas import tpu as pltpu
from jax.experimental.pallas import tpu_sc as plsc

_Q = 0.7
_SEED = 42
_C = 128
_NC = 2
_NS = 16
_NW = _NC * _NS

_plan_cache = {}


def _threefry2x32(k0, k1, x0, x1):
    rot = (13, 15, 26, 6, 17, 29, 16, 24)
    ks = (np.uint32(k0), np.uint32(k1),
          np.uint32(k0) ^ np.uint32(k1) ^ np.uint32(0x1BD11BDA))
    x0 = (x0 + ks[0]).astype(np.uint32)
    x1 = (x1 + ks[1]).astype(np.uint32)
    for i in range(5):
        for r in rot[:4] if i % 2 == 0 else rot[4:]:
            x0 = (x0 + x1).astype(np.uint32)
            x1 = ((x1 << np.uint32(r)) | (x1 >> np.uint32(32 - r))).astype(np.uint32)
            x1 = x1 ^ x0
        x0 = (x0 + ks[(i + 1) % 3]).astype(np.uint32)
        x1 = (x1 + ks[(i + 2) % 3] + np.uint32(i + 1)).astype(np.uint32)
    return x0, x1


def _bernoulli_mask(seed, p, n):
    k0 = np.uint32(np.uint64(seed) >> np.uint64(32))
    k1 = np.uint32(np.uint64(seed) & np.uint64(0xFFFFFFFF))
    idx = np.arange(n, dtype=np.uint64)
    c1 = (idx >> np.uint64(32)).astype(np.uint32)
    c2 = (idx & np.uint64(0xFFFFFFFF)).astype(np.uint32)
    b1, b2 = _threefry2x32(k0, k1, c1, c2)
    bits = b1 ^ b2
    floats = ((bits >> np.uint32(9)) | np.uint32(0x3F800000)).view(np.float32)
    u = np.maximum(np.float32(0), floats - np.float32(1))
    return u < np.float32(p)


def _pack(idx):
    per = -(-len(idx) // _NW)
    per = -(-per // _C) * _C
    pad = np.full(per * _NW - len(idx), idx[0], np.int32)
    return np.concatenate([idx, pad]).reshape(_NW, per // _C, _C)


def _mask_plan(n):
    if n not in _plan_cache:
        mask = _bernoulli_mask(_SEED, _Q, n)
        keep = np.nonzero(~mask)[0].astype(np.int32)
        zero = np.nonzero(mask)[0].astype(np.int32)
        _plan_cache[n] = (_pack(keep), _pack(zero))
    return _plan_cache[n]


def _sc_body(kc, zc, x_hbm, kidx_hbm, zidx_hbm, zeros_hbm, out_hbm,
             kidx_v, zidx_v, rowbuf, zbuf, gsem, ssem, zsem):
    w = lax.axis_index("s") * _NC + lax.axis_index("c")
    pltpu.sync_copy(kidx_hbm.at[w], kidx_v)
    pltpu.sync_copy(zidx_hbm.at[w], zidx_v)
    pltpu.sync_copy(zeros_hbm, zbuf)

    zd = [pltpu.async_copy(zbuf, out_hbm.at[zidx_v.at[j]], zsem)
          for j in range(zc)]

    def gather(j):
        return pltpu.async_copy(x_hbm.at[kidx_v.at[j]], rowbuf.at[j % 2], gsem)

    def scatter(j):
        return pltpu.async_copy(rowbuf.at[j % 2], out_hbm.at[kidx_v.at[j]], ssem)

    gd = {0: gather(0)}
    sd = {}
    for j in range(kc):
        gd[j].wait()
        if j + 1 < kc:
            if j - 1 >= 0:
                sd[j - 1].wait()
            gd[j + 1] = gather(j + 1)
        sd[j] = scatter(j)
    for j in range(max(0, kc - 2), kc):
        sd[j].wait()
    for d in zd:
        d.wait()


def kernel(x):
    n, d = x.shape
    kidx, zidx = _mask_plan(n)
    kc, zc = kidx.shape[1], zidx.shape[1]
    zeros = jnp.zeros((_C, d), x.dtype)
    mesh = plsc.VectorSubcoreMesh(
        core_axis_name="c", subcore_axis_name="s",
        num_cores=_NC, num_subcores=_NS)
    body = functools.partial(_sc_body, kc, zc)
    return pl.kernel(
        body,
        out_type=jax.ShapeDtypeStruct((n, d), x.dtype),
        mesh=mesh,
        scratch_types=[
            pltpu.VMEM((kc, _C), jnp.int32),
            pltpu.VMEM((zc, _C), jnp.int32),
            pltpu.VMEM((2, _C, d), x.dtype),
            pltpu.VMEM((_C, d), x.dtype),
            pltpu.SemaphoreType.DMA,
            pltpu.SemaphoreType.DMA,
            pltpu.SemaphoreType.DMA,
        ],
    )(x, jnp.asarray(kidx), jnp.asarray(zidx), zeros)


# speedup vs baseline: 5.7431x; 3.8985x over previous
import functools
import numpy as np
import jax
import jax.numpy as jnp
from jax import lax
from jax.experimental import pallas as pl
from jax.experimental.pallas import tpu as pltpu
from jax.experimental.pallas import tpu_sc as plsc

_Q = 0.7
_SEED = 42
_C = 128
_NC = 2
_NS = 16
_NW = _NC * _NS

_plan_cache = {}


def _threefry2x32(k0, k1, x0, x1):
    rot = (13, 15, 26, 6, 17, 29, 16, 24)
    ks = (np.uint32(k0), np.uint32(k1),
          np.uint32(k0) ^ np.uint32(k1) ^ np.uint32(0x1BD11BDA))
    x0 = (x0 + ks[0]).astype(np.uint32)
    x1 = (x1 + ks[1]).astype(np.uint32)
    for i in range(5):
        for r in rot[:4] if i % 2 == 0 else rot[4:]:
            x0 = (x0 + x1).astype(np.uint32)
            x1 = ((x1 << np.uint32(r)) | (x1 >> np.uint32(32 - r))).astype(np.uint32)
            x1 = x1 ^ x0
        x0 = (x0 + ks[(i + 1) % 3]).astype(np.uint32)
        x1 = (x1 + ks[(i + 2) % 3] + np.uint32(i + 1)).astype(np.uint32)
    return x0, x1


def _bernoulli_mask(seed, p, n):
    k0 = np.uint32(np.uint64(seed) >> np.uint64(32))
    k1 = np.uint32(np.uint64(seed) & np.uint64(0xFFFFFFFF))
    idx = np.arange(n, dtype=np.uint64)
    c1 = (idx >> np.uint64(32)).astype(np.uint32)
    c2 = (idx & np.uint64(0xFFFFFFFF)).astype(np.uint32)
    b1, b2 = _threefry2x32(k0, k1, c1, c2)
    bits = b1 ^ b2
    floats = ((bits >> np.uint32(9)) | np.uint32(0x3F800000)).view(np.float32)
    u = np.maximum(np.float32(0), floats - np.float32(1))
    return u < np.float32(p)


def _pack(idx):
    per = -(-len(idx) // _NW)
    per = -(-per // _C) * _C
    npad = per * _NW - len(idx)
    reps = -(-npad // len(idx))
    pad = np.tile(idx, reps)[:npad]
    return np.concatenate([idx, pad]).reshape(_NW, per // _C, _C)


def _mask_plan(n):
    if n not in _plan_cache:
        mask = _bernoulli_mask(_SEED, _Q, n)
        keep = np.nonzero(~mask)[0].astype(np.int32)
        zero = np.nonzero(mask)[0].astype(np.int32)
        _plan_cache[n] = (_pack(keep), _pack(zero))
    return _plan_cache[n]


def _sc_body(kc, zc, x_hbm, kidx_hbm, zidx_hbm, zeros_hbm, out_hbm,
             kidx_v, zidx_v, rowbuf, zbuf, gsem, ssem, zsem):
    w = lax.axis_index("s") * _NC + lax.axis_index("c")
    pltpu.sync_copy(kidx_hbm.at[w], kidx_v)
    pltpu.sync_copy(zidx_hbm.at[w], zidx_v)
    pltpu.sync_copy(zeros_hbm.at[w], zbuf)

    zd = [pltpu.async_copy(zbuf, out_hbm.at[zidx_v.at[j]], zsem)
          for j in range(zc)]

    def gather(j):
        return pltpu.async_copy(x_hbm.at[kidx_v.at[j]], rowbuf.at[j % 2], gsem)

    def scatter(j):
        return pltpu.async_copy(rowbuf.at[j % 2], out_hbm.at[kidx_v.at[j]], ssem)

    gd = {0: gather(0)}
    sd = {}
    for j in range(kc):
        gd[j].wait()
        if j + 1 < kc:
            if j - 1 >= 0:
                sd[j - 1].wait()
            gd[j + 1] = gather(j + 1)
        sd[j] = scatter(j)
    for j in range(max(0, kc - 2), kc):
        sd[j].wait()
    for d in zd:
        d.wait()


def kernel(x):
    n, d = x.shape
    kidx, zidx = _mask_plan(n)
    kc, zc = kidx.shape[1], zidx.shape[1]
    zeros = jnp.zeros((_NW, _C, d), x.dtype)
    mesh = plsc.VectorSubcoreMesh(
        core_axis_name="c", subcore_axis_name="s",
        num_cores=_NC, num_subcores=_NS)
    body = functools.partial(_sc_body, kc, zc)
    return pl.kernel(
        body,
        out_type=jax.ShapeDtypeStruct((n, d), x.dtype),
        mesh=mesh,
        scratch_types=[
            pltpu.VMEM((kc, _C), jnp.int32),
            pltpu.VMEM((zc, _C), jnp.int32),
            pltpu.VMEM((2, _C, d), x.dtype),
            pltpu.VMEM((_C, d), x.dtype),
            pltpu.SemaphoreType.DMA,
            pltpu.SemaphoreType.DMA,
            pltpu.SemaphoreType.DMA,
        ],
    )(x, jnp.asarray(kidx), jnp.asarray(zidx), zeros)
